# baseline (speedup 1.0000x reference)
import jax
import jax.numpy as jnp
from jax import lax
from jax.experimental import pallas as pl
from jax.experimental.pallas import tpu as pltpu
from jax.experimental.pallas import tpu_sc as plsc

N_EMBD = 1024
N_FEATURES = 131072
S = 4096
K = 16
NC, NS, L = 2, 16, 16
NW = NC * NS
TOK_PER_W = S // NW
T = 2
NCHUNK = TOK_PER_W // T
ROWS = T * K
NCOL = N_EMBD // L


def _sae_body(vals_hbm, idx_hbm, wdec_hbm, bdec_hbm, out_hbm,
              vals_v, idx_v, bdec_v, rows_v, ostage_v,
              sem0, sem1, osem0, osem1):
    wid = lax.axis_index("s") * NC + lax.axis_index("c")
    tok0 = wid * TOK_PER_W
    base = tok0 * K

    pltpu.sync_copy(vals_hbm.at[pl.ds(base, TOK_PER_W * K)], vals_v)
    pltpu.sync_copy(idx_hbm.at[pl.ds(base, TOK_PER_W * K)], idx_v)
    pltpu.sync_copy(bdec_hbm, bdec_v)

    sems = (sem0, sem1)

    def gather(c, buf):
        return pltpu.make_async_copy(
            wdec_hbm.at[idx_v.at[pl.ds(c * ROWS, ROWS)]],
            rows_v.at[buf], sems[buf])

    osems = (osem0, osem1)

    def out_store(c, buf):
        return pltpu.make_async_copy(
            ostage_v.at[buf], out_hbm.at[pl.ds(tok0 + c * T, T)], osems[buf])

    def compute(i, c, buf):
        off = c * ROWS
        ws = []
        for t in range(T):
            vv = vals_v[pl.ds(off + t * K, K)]
            ws.extend(vv[k] for k in range(K))
        gather(c, buf).wait()

        @pl.when(i > 0)
        def _():
            out_store(c, buf).wait()

        @plsc.parallel_loop(0, NCOL, unroll=2)
        def col_body(j):
            b = bdec_v[pl.ds(j * L, L)]
            for t in range(T):
                acc = b
                for k in range(K):
                    acc = acc + ws[t * K + k] * rows_v[buf, t * K + k,
                                                      pl.ds(j * L, L)]
                ostage_v[buf, t, pl.ds(j * L, L)] = acc
        out_store(c, buf).start()

    gather(0, 0).start()

    def pair_body(i, carry):
        c0 = 2 * i
        gather(c0 + 1, 1).start()
        compute(i, c0, 0)

        @pl.when(i < NCHUNK // 2 - 1)
        def _():
            gather(c0 + 2, 0).start()

        compute(i, c0 + 1, 1)
        return carry

    lax.fori_loop(0, NCHUNK // 2, pair_body, 0, unroll=False)
    out_store(NCHUNK - 2, 0).wait()
    out_store(NCHUNK - 1, 1).wait()


@jax.jit
def _sae_decode(vals, idx, W_dec, b_dec):
    mesh = plsc.VectorSubcoreMesh(
        core_axis_name="c", subcore_axis_name="s",
        num_cores=NC, num_subcores=NS)
    return pl.kernel(
        _sae_body,
        out_type=jax.ShapeDtypeStruct((S, N_EMBD), jnp.float32),
        mesh=mesh,
        scratch_types=[
            pltpu.VMEM((TOK_PER_W * K,), jnp.float32),
            pltpu.VMEM((TOK_PER_W * K,), jnp.int32),
            pltpu.VMEM((N_EMBD,), jnp.float32),
            pltpu.VMEM((2, ROWS, N_EMBD), jnp.float32),
            pltpu.VMEM((2, T, N_EMBD), jnp.float32),
            pltpu.SemaphoreType.DMA,
            pltpu.SemaphoreType.DMA,
            pltpu.SemaphoreType.DMA,
            pltpu.SemaphoreType.DMA,
        ],
    )(vals, idx, W_dec, b_dec)


def kernel(feat_values, feat_indices, W_dec, b_dec):
    vals = feat_values.reshape(-1).astype(jnp.float32)
    idx = feat_indices.reshape(-1).astype(jnp.int32)
    out = _sae_decode(vals, idx, W_dec, b_dec)
    probe = jnp.minimum(jnp.abs(jnp.sum(W_dec)), 0.0)
    return (out + probe).reshape(1, S, N_EMBD)


# speedup vs baseline: 2.0932x; 2.0932x over previous
import jax
import jax.numpy as jnp
from jax import lax
from jax.experimental import pallas as pl
from jax.experimental.pallas import tpu as pltpu
from jax.experimental.pallas import tpu_sc as plsc

N_EMBD = 1024
N_FEATURES = 131072
S = 4096
K = 16
NC, NS, L = 2, 16, 16
NW = NC * NS
TOK_PER_W = S // NW
T = 2
NCHUNK = TOK_PER_W // T
ROWS = T * K
NCOL = N_EMBD // L


def _sae_body(vals_hbm, idx_hbm, wdec_hbm, bdec_hbm, out_hbm,
              vals_v, idx_v, bdec_v, rows_v, ostage_v,
              sem0, sem1, osem0, osem1):
    wid = lax.axis_index("s") * NC + lax.axis_index("c")
    tok0 = wid * TOK_PER_W
    base = tok0 * K

    sems = (sem0, sem1)

    st_i = pltpu.make_async_copy(
        idx_hbm.at[pl.ds(base, TOK_PER_W * K)], idx_v, sem0)
    st_v = pltpu.make_async_copy(
        vals_hbm.at[pl.ds(base, TOK_PER_W * K)], vals_v, sem1)
    st_b = pltpu.make_async_copy(bdec_hbm, bdec_v, osem0)
    st_i.start()
    st_v.start()
    st_b.start()

    def gather(c, buf):
        return pltpu.make_async_copy(
            wdec_hbm.at[idx_v.at[pl.ds(c * ROWS, ROWS)]],
            rows_v.at[buf], sems[buf])

    osems = (osem0, osem1)

    def out_store(c, buf):
        return pltpu.make_async_copy(
            ostage_v.at[buf], out_hbm.at[pl.ds(tok0 + c * T, T)], osems[buf])

    def compute(i, c, buf):
        off = c * ROWS
        ws = []
        for t in range(T):
            vv = vals_v[pl.ds(off + t * K, K)]
            ws.extend(vv[k] for k in range(K))
        gather(c, buf).wait()

        @pl.when(i > 0)
        def _():
            out_store(c, buf).wait()

        @plsc.parallel_loop(0, NCOL, unroll=2)
        def col_body(j):
            b = bdec_v[pl.ds(j * L, L)]
            for t in range(T):
                acc = b
                for k in range(K):
                    acc = acc + ws[t * K + k] * rows_v[buf, t * K + k,
                                                      pl.ds(j * L, L)]
                ostage_v[buf, t, pl.ds(j * L, L)] = acc
        out_store(c, buf).start()

    st_i.wait()
    gather(0, 0).start()
    st_v.wait()
    st_b.wait()

    def pair_body(i, carry):
        c0 = 2 * i
        gather(c0 + 1, 1).start()
        compute(i, c0, 0)

        @pl.when(i < NCHUNK // 2 - 1)
        def _():
            gather(c0 + 2, 0).start()

        compute(i, c0 + 1, 1)
        return carry

    lax.fori_loop(0, NCHUNK // 2, pair_body, 0, unroll=False)
    out_store(NCHUNK - 2, 0).wait()
    out_store(NCHUNK - 1, 1).wait()


@jax.jit
def _sae_decode(vals, idx, W_dec, b_dec):
    mesh = plsc.VectorSubcoreMesh(
        core_axis_name="c", subcore_axis_name="s",
        num_cores=NC, num_subcores=NS)
    return pl.kernel(
        _sae_body,
        out_type=jax.ShapeDtypeStruct((S, N_EMBD), jnp.float32),
        mesh=mesh,
        scratch_types=[
            pltpu.VMEM((TOK_PER_W * K,), jnp.float32),
            pltpu.VMEM((TOK_PER_W * K,), jnp.int32),
            pltpu.VMEM((N_EMBD,), jnp.float32),
            pltpu.VMEM((2, ROWS, N_EMBD), jnp.float32),
            pltpu.VMEM((2, T, N_EMBD), jnp.float32),
            pltpu.SemaphoreType.DMA,
            pltpu.SemaphoreType.DMA,
            pltpu.SemaphoreType.DMA,
            pltpu.SemaphoreType.DMA,
        ],
    )(vals, idx, W_dec, b_dec)


def kernel(feat_values, feat_indices, W_dec, b_dec):
    vals = feat_values.reshape(-1).astype(jnp.float32)
    idx = feat_indices.reshape(-1).astype(jnp.int32)
    out = _sae_decode(vals, idx, W_dec, b_dec)
    return out.reshape(1, S, N_EMBD)


# speedup vs baseline: 2.0936x; 1.0002x over previous
import jax
import jax.numpy as jnp
from jax import lax
from jax.experimental import pallas as pl
from jax.experimental.pallas import tpu as pltpu
from jax.experimental.pallas import tpu_sc as plsc

N_EMBD = 1024
N_FEATURES = 131072
S = 4096
K = 16
NC, NS, L = 2, 16, 16
NW = NC * NS
TOK_PER_W = S // NW
T = 2
NCHUNK = TOK_PER_W // T
ROWS = T * K
NCOL = N_EMBD // L


def _sae_body(vals_hbm, idx_hbm, wdec_hbm, bdec_hbm, out_hbm,
              vals_v, idx_v, bdec_v, rows_v, ostage_v,
              sem0, sem1, osem0, osem1):
    wid = lax.axis_index("s") * NC + lax.axis_index("c")
    tok0 = wid * TOK_PER_W
    base = tok0 * K

    sems = (sem0, sem1)

    st_i = pltpu.make_async_copy(
        idx_hbm.at[pl.ds(base, TOK_PER_W * K)], idx_v, sem0)
    st_v = pltpu.make_async_copy(
        vals_hbm.at[pl.ds(base, TOK_PER_W * K)], vals_v, sem1)
    st_b = pltpu.make_async_copy(bdec_hbm, bdec_v, osem0)
    st_i.start()
    st_v.start()
    st_b.start()

    def gather(c, buf):
        return pltpu.make_async_copy(
            wdec_hbm.at[idx_v.at[pl.ds(c * ROWS, ROWS)]],
            rows_v.at[buf], sems[buf])

    osems = (osem0, osem1)

    def out_store(c, buf):
        return pltpu.make_async_copy(
            ostage_v.at[buf], out_hbm.at[pl.ds(tok0 + c * T, T)], osems[buf])

    def compute(i, c, buf):
        off = c * ROWS
        ws = []
        for t in range(T):
            vv = vals_v[pl.ds(off + t * K, K)]
            ws.extend(vv[k] for k in range(K))
        gather(c, buf).wait()

        @pl.when(i > 0)
        def _():
            out_store(c, buf).wait()

        @plsc.parallel_loop(0, NCOL, unroll=3)
        def col_body(j):
            b = bdec_v[pl.ds(j * L, L)]
            for t in range(T):
                acc = b
                for k in range(K):
                    acc = acc + ws[t * K + k] * rows_v[buf, t * K + k,
                                                      pl.ds(j * L, L)]
                ostage_v[buf, t, pl.ds(j * L, L)] = acc
        out_store(c, buf).start()

    st_i.wait()
    gather(0, 0).start()
    st_v.wait()
    st_b.wait()

    def pair_body(i, carry):
        c0 = 2 * i
        gather(c0 + 1, 1).start()
        compute(i, c0, 0)

        @pl.when(i < NCHUNK // 2 - 1)
        def _():
            gather(c0 + 2, 0).start()

        compute(i, c0 + 1, 1)
        return carry

    lax.fori_loop(0, NCHUNK // 2, pair_body, 0, unroll=False)
    out_store(NCHUNK - 2, 0).wait()
    out_store(NCHUNK - 1, 1).wait()


@jax.jit
def _sae_decode(vals, idx, W_dec, b_dec):
    mesh = plsc.VectorSubcoreMesh(
        core_axis_name="c", subcore_axis_name="s",
        num_cores=NC, num_subcores=NS)
    return pl.kernel(
        _sae_body,
        out_type=jax.ShapeDtypeStruct((S, N_EMBD), jnp.float32),
        mesh=mesh,
        scratch_types=[
            pltpu.VMEM((TOK_PER_W * K,), jnp.float32),
            pltpu.VMEM((TOK_PER_W * K,), jnp.int32),
            pltpu.VMEM((N_EMBD,), jnp.float32),
            pltpu.VMEM((2, ROWS, N_EMBD), jnp.float32),
            pltpu.VMEM((2, T, N_EMBD), jnp.float32),
            pltpu.SemaphoreType.DMA,
            pltpu.SemaphoreType.DMA,
            pltpu.SemaphoreType.DMA,
            pltpu.SemaphoreType.DMA,
        ],
    )(vals, idx, W_dec, b_dec)


def kernel(feat_values, feat_indices, W_dec, b_dec):
    vals = feat_values.reshape(-1).astype(jnp.float32)
    idx = feat_indices.reshape(-1).astype(jnp.int32)
    out = _sae_decode(vals, idx, W_dec, b_dec)
    return out.reshape(1, S, N_EMBD)
